# baseline (speedup 1.0000x reference)
import functools

import jax
import jax.numpy as jnp
from jax import lax
from jax.experimental import pallas as pl
from jax.experimental.pallas import tpu as pltpu
from jax.experimental.pallas import tpu_sc as plsc

_B, _P, _D, _OUT = 4096, 32, 128, 128

_NC = 2
_NS = 16
_NW = _NC * _NS
_EPW = _B // _NW
_CH = 8
_NCH = _EPW // _CH
_ROWS = _CH * _P
_NVR = _D // 16


def _sc_reduce_body(x_hbm, s_hbm, buf, acc, sem0, sem1):
    c = lax.axis_index("c")
    s = lax.axis_index("s")
    wid = s * _NC + c
    row0 = wid * _EPW * _P
    sems = (sem0, sem1)

    def start(g):
        slot = g % 2
        return pltpu.async_copy(
            x_hbm.at[pl.ds(row0 + g * _ROWS, _ROWS)], buf.at[slot], sems[slot])

    def chunk_compute(slot, erow0):
        def ebody(e, _):
            def pbody(p, vs):
                r = e * _P + p * 4
                out = []
                for j in range(_NVR):
                    v = vs[j]
                    for u in range(4):
                        v = v + buf[slot, r + u, pl.ds(16 * j, 16)]
                    out.append(v)
                return tuple(out)

            zeros = tuple(jnp.zeros((16,), jnp.float32) for _ in range(_NVR))
            vs = lax.fori_loop(0, _P // 4, pbody, zeros)
            for j in range(_NVR):
                acc[erow0 + e, pl.ds(16 * j, 16)] = vs[j]
            return 0

        lax.fori_loop(0, _CH, ebody, 0)

    handles = [start(0), start(1)]
    for g in range(_NCH):
        slot = g % 2
        handles[slot].wait()
        chunk_compute(slot, g * _CH)
        if g + 2 < _NCH:
            handles[slot] = start(g + 2)
    pltpu.sync_copy(acc, s_hbm.at[pl.ds(wid * _EPW, _EPW)])


_sc_reduce = functools.partial(
    pl.kernel,
    _sc_reduce_body,
    out_type=jax.ShapeDtypeStruct((_B, _D), jnp.float32),
    scratch_types=[
        pltpu.VMEM((2, _ROWS, _D), jnp.float32),
        pltpu.VMEM((_EPW, _D), jnp.float32),
        pltpu.SemaphoreType.DMA,
        pltpu.SemaphoreType.DMA,
    ],
    mesh=plsc.VectorSubcoreMesh(core_axis_name="c", subcore_axis_name="s"),
)()


def _mm_body(s_ref, w_ref, b_ref, o_ref):
    o_ref[...] = jax.lax.dot_general(
        s_ref[...], w_ref[...], (((1,), (1,)), ((), ())),
        preferred_element_type=jnp.float32,
        precision=jax.lax.Precision.HIGHEST,
    ) + b_ref[...]


def _tc_matmul(sums, W, bscaled, blk):
    n = sums.shape[0]
    return pl.pallas_call(
        _mm_body,
        grid=(n // blk,),
        in_specs=[
            pl.BlockSpec((blk, _D), lambda i: (i, 0)),
            pl.BlockSpec((_OUT, _D), lambda i: (0, 0)),
            pl.BlockSpec((1, _OUT), lambda i: (0, 0)),
        ],
        out_specs=pl.BlockSpec((blk, _OUT), lambda i: (i, 0)),
        out_shape=jax.ShapeDtypeStruct((n, _OUT), jnp.float32),
    )(sums, W, bscaled)


def kernel(pert_batch, W, b):
    bscaled = (float(_P) * b).reshape(1, _OUT)
    flat = pert_batch.reshape(_B * _P, _D)
    sums = _sc_reduce(flat)
    return _tc_matmul(sums, W, bscaled, 512)
